# Initial kernel scaffold; baseline (speedup 1.0000x reference)
import jax
import jax.numpy as jnp
from jax.experimental import pallas as pl


def kernel(inputs, grids):
    raise NotImplementedError("write your pallas kernel here")


# speedup vs baseline: 4.2795x; 4.2795x over previous
import functools
import math

import numpy as np

import jax
import jax.numpy as jnp
from jax import lax
from jax.experimental import pallas as pl
from jax.experimental.pallas import tpu as pltpu
from jax.experimental.pallas import tpu_sc as plsc

_N_LEVELS = 16
_N_FEAT = 2
_ROW = 8
_LOG2_HASHMAP = 19
_BASE_RES = 16
_PER_LEVEL_SCALE = 1.38191288

_N = 262144
_NC = 2
_NS = 16
_NW = _NC * _NS
_C = 64
_NG = _C // 16
_PTS_PER_W = _N // _NW
_CHUNKS = _PTS_PER_W // _C
_OUT_W = 2 * _N_LEVELS

_P2 = np.int32(-1640531535)
_P3 = np.int32(805459861)

_CORNERS = [(ox, oy, oz) for ox in (0, 1) for oy in (0, 1) for oz in (0, 1)]


def _level_params():
    thr = 1 << _LOG2_HASHMAP
    params = []
    for i in range(_N_LEVELS):
        scale = math.pow(2.0, i * math.log2(_PER_LEVEL_SCALE)) * _BASE_RES - 1.0
        res = math.ceil(scale) + 1
        size = min(math.ceil(res ** 3 / 8) * 8, thr)
        params.append((scale, size, res, size >= thr))
    return params


_LVL = _level_params()


def _gather_rows(table, idx_ref, dst, sem):
    return pltpu.async_copy(table.at[idx_ref], dst, sem)


def _vgather(ref, idxs):
    return plsc.load_gather(ref, idxs)


def _vscatter(ref, idxs, x):
    plsc.store_scatter(ref, idxs, x)


def _worker_id():
    return lax.axis_index("s") * _NC + lax.axis_index("c")


def _body(xs_h, ys_h, zs_h, *rest):
    tables = rest[:_N_LEVELS]
    out_h = rest[_N_LEVELS]
    (xv, yv, zv, dxb, dyb, dzb, idxb, rowsb, outc, sem) = rest[_N_LEVELS + 1:]

    wid = _worker_id()
    base0 = wid * _PTS_PER_W
    iota = lax.iota(jnp.int32, 16)
    zero16 = jnp.zeros((16,), jnp.int32)
    one16 = jnp.ones((16,), jnp.int32)

    def chunk_body(ci, carry):
        base = base0 + ci * _C
        pltpu.sync_copy(xs_h.at[pl.ds(base, _C)], xv)
        pltpu.sync_copy(ys_h.at[pl.ds(base, _C)], yv)
        pltpu.sync_copy(zs_h.at[pl.ds(base, _C)], zv)

        copies = []
        for lvl in range(_N_LEVELS):
            scale, size, res, is_hash = _LVL[lvl]
            fscale = jnp.float32(scale)

            def body_a(g, c2, lvl=lvl, fscale=fscale, size=size, res=res,
                       is_hash=is_hash):
                off = g * 16
                xw = xv[pl.ds(off, 16)] * fscale + jnp.float32(0.5)
                yw = yv[pl.ds(off, 16)] * fscale + jnp.float32(0.5)
                zw = zv[pl.ds(off, 16)] * fscale + jnp.float32(0.5)
                xg = xw.astype(jnp.int32)
                yg = yw.astype(jnp.int32)
                zg = zw.astype(jnp.int32)
                dxb[lvl, pl.ds(off, 16)] = xw - xg.astype(jnp.float32)
                dyb[lvl, pl.ds(off, 16)] = yw - yg.astype(jnp.float32)
                dzb[lvl, pl.ds(off, 16)] = zw - zg.astype(jnp.float32)
                if is_hash:
                    mask = jnp.int32(size - 1)
                    hx = (xg, xg + jnp.int32(1))
                    hy0 = yg * _P2
                    hz0 = zg * _P3
                    hy = (hy0, hy0 + _P2)
                    hz = (hz0, hz0 + _P3)
                    for c, (ox, oy, oz) in enumerate(_CORNERS):
                        idxb[lvl, c, pl.ds(off, 16)] = (hx[ox] ^ hy[oy] ^ hz[oz]) & mask
                else:
                    s = jnp.int32(res)
                    s2 = jnp.int32(res * res)
                    tx = (xg, xg + jnp.int32(1))
                    ty0 = yg * s
                    tz0 = zg * s2
                    ty = (ty0, ty0 + s)
                    tz = (tz0, tz0 + s2)
                    sz = jnp.int32(size)
                    for c, (ox, oy, oz) in enumerate(_CORNERS):
                        idx = tx[ox] + ty[oy] + tz[oz]
                        idxb[lvl, c, pl.ds(off, 16)] = jnp.maximum(lax.rem(idx, sz), 0)
                return c2

            lax.fori_loop(0, _NG, body_a, 0)
            for c in range(8):
                copies.append(
                    _gather_rows(tables[lvl], idxb.at[lvl, c],
                                 rowsb.at[lvl, c], sem))
        for cp in copies:
            cp.wait()

        for lvl in range(_N_LEVELS):
            def body_b(g, c2, lvl=lvl):
                off = g * 16
                dx = dxb[lvl, pl.ds(off, 16)]
                dy = dyb[lvl, pl.ds(off, 16)]
                dz = dzb[lvl, pl.ds(off, 16)]
                one = jnp.float32(1.0)
                wx = (one - dx, dx)
                wy = (one - dy, dy)
                wz = (one - dz, dz)
                wxy = ((wx[0] * wy[0], wx[0] * wy[1]),
                       (wx[1] * wy[0], wx[1] * wy[1]))
                rows = iota + off
                acc0 = jnp.zeros((16,), jnp.float32)
                acc1 = jnp.zeros((16,), jnp.float32)
                for c, (ox, oy, oz) in enumerate(_CORNERS):
                    w = wxy[ox][oy] * wz[oz]
                    f0 = _vgather(rowsb.at[lvl, c], [rows, zero16])
                    f1 = _vgather(rowsb.at[lvl, c], [rows, one16])
                    acc0 = acc0 + f0 * w
                    acc1 = acc1 + f1 * w
                flat0 = rows * jnp.int32(_OUT_W) + jnp.int32(2 * lvl)
                _vscatter(outc, [flat0], acc0)
                _vscatter(outc, [flat0 + one16], acc1)
                return c2

            lax.fori_loop(0, _NG, body_b, 0)

        pltpu.sync_copy(outc, out_h.at[pl.ds(base * _OUT_W, _C * _OUT_W)])
        return carry

    lax.fori_loop(0, _CHUNKS, chunk_body, 0)


_encode = functools.partial(
    pl.kernel,
    out_type=jax.ShapeDtypeStruct((_N * _OUT_W,), jnp.float32),
    mesh=plsc.VectorSubcoreMesh(core_axis_name="c", subcore_axis_name="s",
                                num_cores=_NC, num_subcores=_NS),
    compiler_params=pltpu.CompilerParams(needs_layout_passes=False,
                                         use_tc_tiling_on_sc=False),
    scratch_types=[
        pltpu.VMEM((_C,), jnp.float32),
        pltpu.VMEM((_C,), jnp.float32),
        pltpu.VMEM((_C,), jnp.float32),
        pltpu.VMEM((_N_LEVELS, _C), jnp.float32),
        pltpu.VMEM((_N_LEVELS, _C), jnp.float32),
        pltpu.VMEM((_N_LEVELS, _C), jnp.float32),
        pltpu.VMEM((_N_LEVELS, 8, _C), jnp.int32),
        pltpu.VMEM((_N_LEVELS, 8, _C, _ROW), jnp.float32),
        pltpu.VMEM((_C * _OUT_W,), jnp.float32),
        pltpu.SemaphoreType.DMA,
    ],
)(_body)


def kernel(inputs, grids):
    xs = inputs[:, 0]
    ys = inputs[:, 1]
    zs = inputs[:, 2]
    padded = tuple(
        jnp.pad(g, ((0, 0), (0, _ROW - _N_FEAT))) for g in grids
    )
    flat = _encode(xs, ys, zs, *padded)
    return flat.reshape(_N, _OUT_W)


# speedup vs baseline: 5.4361x; 1.2702x over previous
import functools
import math

import numpy as np

import jax
import jax.numpy as jnp
from jax import lax
from jax.experimental import pallas as pl
from jax.experimental.pallas import tpu as pltpu
from jax.experimental.pallas import tpu_sc as plsc

_N_LEVELS = 16
_N_FEAT = 2
_ROW = 8
_LOG2_HASHMAP = 19
_BASE_RES = 16
_PER_LEVEL_SCALE = 1.38191288

_N = 262144
_NC = 2
_NS = 16
_NW = _NC * _NS
_C = 64
_NG = _C // 16
_PTS_PER_W = _N // _NW
_CHUNKS = _PTS_PER_W // _C
_OUT_W = 2 * _N_LEVELS

_P2 = np.int32(-1640531535)
_P3 = np.int32(805459861)

_CORNERS = [(ox, oy, oz) for ox in (0, 1) for oy in (0, 1) for oz in (0, 1)]


def _level_params():
    thr = 1 << _LOG2_HASHMAP
    params = []
    for i in range(_N_LEVELS):
        scale = math.pow(2.0, i * math.log2(_PER_LEVEL_SCALE)) * _BASE_RES - 1.0
        res = math.ceil(scale) + 1
        size = min(math.ceil(res ** 3 / 8) * 8, thr)
        params.append((scale, size, res, size >= thr))
    return params


_LVL = _level_params()


_PCH = 2048


def _pad_body(*args):
    srcs = args[:_N_LEVELS]
    outs = args[_N_LEVELS:2 * _N_LEVELS]
    stage, chunk8, = args[2 * _N_LEVELS:]
    wid = _worker_id()
    iota = lax.iota(jnp.int32, 16)
    rowi = lax.shift_right_logical(iota, 1)
    coli = lax.bitwise_and(iota, jnp.int32(1))

    def expand(nrows, base_rows):
        def g_body(g, c2):
            v = stage[pl.ds(g * 16, 16)]
            _vscatter(chunk8, [rowi + g * 8, coli], v)
            return c2
        lax.fori_loop(0, nrows // 8, g_body, 0)

    for lvl in range(_N_LEVELS):
        size = _LVL[lvl][1]
        full = size // _PCH
        tail = size - full * _PCH
        src, out = srcs[lvl], outs[lvl]
        if full:
            def k_body(k, c2, src=src, out=out, full=full):
                cid = k * _NW + wid

                @pl.when(cid < full)
                def _():
                    r0 = cid * _PCH
                    pltpu.sync_copy(src.at[pl.ds(r0 * 2, _PCH * 2)], stage)
                    expand(_PCH, r0)
                    pltpu.sync_copy(chunk8, out.at[pl.ds(r0, _PCH), :])
                return c2
            lax.fori_loop(0, -(-full // _NW), k_body, 0)
        if tail:
            @pl.when(wid == jnp.int32(lvl % _NW))
            def _(src=src, out=out, full=full, tail=tail):
                r0 = full * _PCH
                pltpu.sync_copy(src.at[pl.ds(r0 * 2, tail * 2)],
                                stage.at[pl.ds(0, tail * 2)])
                def g_body(g, c2):
                    v = stage[pl.ds(g * 16, 16)]
                    _vscatter(chunk8, [rowi + g * 8, coli], v)
                    return c2
                lax.fori_loop(0, tail // 8, g_body, 0)
                pltpu.sync_copy(chunk8.at[pl.ds(0, tail), :],
                                out.at[pl.ds(r0, tail), :])


_padder = functools.partial(
    pl.kernel,
    out_type=tuple(
        jax.ShapeDtypeStruct((_LVL[i][1], _ROW), jnp.float32)
        for i in range(_N_LEVELS)
    ),
    mesh=plsc.VectorSubcoreMesh(core_axis_name="c", subcore_axis_name="s",
                                num_cores=_NC, num_subcores=_NS),
    compiler_params=pltpu.CompilerParams(needs_layout_passes=False,
                                         use_tc_tiling_on_sc=False),
    scratch_types=[
        pltpu.VMEM((_PCH * 2,), jnp.float32),
        pltpu.VMEM((_PCH, _ROW), jnp.float32),
    ],
)(_pad_body)


def _gather_rows(table, idx_ref, dst, sem):
    return pltpu.async_copy(table.at[idx_ref], dst, sem)


def _vgather(ref, idxs):
    return plsc.load_gather(ref, idxs)


def _vscatter(ref, idxs, x):
    plsc.store_scatter(ref, idxs, x)


def _worker_id():
    return lax.axis_index("s") * _NC + lax.axis_index("c")


def _body(xs_h, ys_h, zs_h, *rest):
    tables = rest[:_N_LEVELS]
    out_h = rest[_N_LEVELS]
    (xv, yv, zv, dxb, dyb, dzb, idxb, rowsb, outc, sem) = rest[_N_LEVELS + 1:]

    wid = _worker_id()
    base0 = wid * _PTS_PER_W
    iota = lax.iota(jnp.int32, 16)
    zero16 = jnp.zeros((16,), jnp.int32)
    one16 = jnp.ones((16,), jnp.int32)

    def chunk_body(ci, carry):
        base = base0 + ci * _C
        pltpu.sync_copy(xs_h.at[pl.ds(base, _C)], xv)
        pltpu.sync_copy(ys_h.at[pl.ds(base, _C)], yv)
        pltpu.sync_copy(zs_h.at[pl.ds(base, _C)], zv)

        copies = []
        for lvl in range(_N_LEVELS):
            scale, size, res, is_hash = _LVL[lvl]
            fscale = jnp.float32(scale)

            def body_a(g, c2, lvl=lvl, fscale=fscale, size=size, res=res,
                       is_hash=is_hash):
                off = g * 16
                xw = xv[pl.ds(off, 16)] * fscale + jnp.float32(0.5)
                yw = yv[pl.ds(off, 16)] * fscale + jnp.float32(0.5)
                zw = zv[pl.ds(off, 16)] * fscale + jnp.float32(0.5)
                xg = xw.astype(jnp.int32)
                yg = yw.astype(jnp.int32)
                zg = zw.astype(jnp.int32)
                dxb[lvl, pl.ds(off, 16)] = xw - xg.astype(jnp.float32)
                dyb[lvl, pl.ds(off, 16)] = yw - yg.astype(jnp.float32)
                dzb[lvl, pl.ds(off, 16)] = zw - zg.astype(jnp.float32)
                if is_hash:
                    mask = jnp.int32(size - 1)
                    hx = (xg, xg + jnp.int32(1))
                    hy0 = yg * _P2
                    hz0 = zg * _P3
                    hy = (hy0, hy0 + _P2)
                    hz = (hz0, hz0 + _P3)
                    for c, (ox, oy, oz) in enumerate(_CORNERS):
                        idxb[lvl, c, pl.ds(off, 16)] = (hx[ox] ^ hy[oy] ^ hz[oz]) & mask
                else:
                    s = jnp.int32(res)
                    s2 = jnp.int32(res * res)
                    tx = (xg, xg + jnp.int32(1))
                    ty0 = yg * s
                    tz0 = zg * s2
                    ty = (ty0, ty0 + s)
                    tz = (tz0, tz0 + s2)
                    sz = jnp.int32(size)
                    for c, (ox, oy, oz) in enumerate(_CORNERS):
                        idx = tx[ox] + ty[oy] + tz[oz]
                        idxb[lvl, c, pl.ds(off, 16)] = jnp.maximum(lax.rem(idx, sz), 0)
                return c2

            lax.fori_loop(0, _NG, body_a, 0)
            for c in range(8):
                copies.append(
                    _gather_rows(tables[lvl], idxb.at[lvl, c],
                                 rowsb.at[lvl, c], sem))
        for cp in copies:
            cp.wait()

        for lvl in range(_N_LEVELS):
            def body_b(g, c2, lvl=lvl):
                off = g * 16
                dx = dxb[lvl, pl.ds(off, 16)]
                dy = dyb[lvl, pl.ds(off, 16)]
                dz = dzb[lvl, pl.ds(off, 16)]
                one = jnp.float32(1.0)
                wx = (one - dx, dx)
                wy = (one - dy, dy)
                wz = (one - dz, dz)
                wxy = ((wx[0] * wy[0], wx[0] * wy[1]),
                       (wx[1] * wy[0], wx[1] * wy[1]))
                rows = iota + off
                acc0 = jnp.zeros((16,), jnp.float32)
                acc1 = jnp.zeros((16,), jnp.float32)
                for c, (ox, oy, oz) in enumerate(_CORNERS):
                    w = wxy[ox][oy] * wz[oz]
                    f0 = _vgather(rowsb.at[lvl, c], [rows, zero16])
                    f1 = _vgather(rowsb.at[lvl, c], [rows, one16])
                    acc0 = acc0 + f0 * w
                    acc1 = acc1 + f1 * w
                flat0 = rows * jnp.int32(_OUT_W) + jnp.int32(2 * lvl)
                _vscatter(outc, [flat0], acc0)
                _vscatter(outc, [flat0 + one16], acc1)
                return c2

            lax.fori_loop(0, _NG, body_b, 0)

        pltpu.sync_copy(outc, out_h.at[pl.ds(base * _OUT_W, _C * _OUT_W)])
        return carry

    lax.fori_loop(0, _CHUNKS, chunk_body, 0)


_encode = functools.partial(
    pl.kernel,
    out_type=jax.ShapeDtypeStruct((_N * _OUT_W,), jnp.float32),
    mesh=plsc.VectorSubcoreMesh(core_axis_name="c", subcore_axis_name="s",
                                num_cores=_NC, num_subcores=_NS),
    compiler_params=pltpu.CompilerParams(needs_layout_passes=False,
                                         use_tc_tiling_on_sc=False),
    scratch_types=[
        pltpu.VMEM((_C,), jnp.float32),
        pltpu.VMEM((_C,), jnp.float32),
        pltpu.VMEM((_C,), jnp.float32),
        pltpu.VMEM((_N_LEVELS, _C), jnp.float32),
        pltpu.VMEM((_N_LEVELS, _C), jnp.float32),
        pltpu.VMEM((_N_LEVELS, _C), jnp.float32),
        pltpu.VMEM((_N_LEVELS, 8, _C), jnp.int32),
        pltpu.VMEM((_N_LEVELS, 8, _C, _ROW), jnp.float32),
        pltpu.VMEM((_C * _OUT_W,), jnp.float32),
        pltpu.SemaphoreType.DMA,
    ],
)(_body)


def kernel(inputs, grids):
    xs = inputs[:, 0]
    ys = inputs[:, 1]
    zs = inputs[:, 2]
    padded = _padder(*(g.reshape(-1) for g in grids))
    flat = _encode(xs, ys, zs, *padded)
    return flat.reshape(_N, _OUT_W)


# speedup vs baseline: 11.4160x; 2.1000x over previous
import functools
import math

import numpy as np

import jax
import jax.numpy as jnp
from jax import lax
from jax.experimental import pallas as pl
from jax.experimental.pallas import tpu as pltpu
from jax.experimental.pallas import tpu_sc as plsc

_N_LEVELS = 16
_N_FEAT = 2
_ROW = 8
_LOG2_HASHMAP = 19
_BASE_RES = 16
_PER_LEVEL_SCALE = 1.38191288

_N = 262144
_NC = 2
_NS = 16
_NW = _NC * _NS
_C = 64
_NG = _C // 16
_PTS_PER_W = _N // _NW
_CHUNKS = _PTS_PER_W // _C
_OUT_W = 2 * _N_LEVELS

_P2 = np.int32(-1640531535)
_P3 = np.int32(805459861)

_CORNERS = [(ox, oy, oz) for ox in (0, 1) for oy in (0, 1) for oz in (0, 1)]


def _level_params():
    thr = 1 << _LOG2_HASHMAP
    params = []
    for i in range(_N_LEVELS):
        scale = math.pow(2.0, i * math.log2(_PER_LEVEL_SCALE)) * _BASE_RES - 1.0
        res = math.ceil(scale) + 1
        size = min(math.ceil(res ** 3 / 8) * 8, thr)
        params.append((scale, size, res, size >= thr))
    return params


_LVL = _level_params()


_PCH = 2048


def _pad_body(*args):
    srcs = args[:_N_LEVELS]
    outs = args[_N_LEVELS:2 * _N_LEVELS]
    stage, stage3, chunk8, = args[2 * _N_LEVELS:]
    wid = _worker_id()
    iota = lax.iota(jnp.int32, 16)
    rowi = lax.shift_right_logical(iota, 1)
    coli = lax.bitwise_and(iota, jnp.int32(1))
    zero16 = jnp.zeros((16,), jnp.int32)
    one16 = jnp.ones((16,), jnp.int32)
    nb = _PCH // 128

    for lvl in range(_N_LEVELS):
        size = _LVL[lvl][1]
        full = size // _PCH
        tail = size - full * _PCH
        src, out = srcs[lvl], outs[lvl]
        if size % 128 == 0:
            assert tail == 0 and full > 0
            def k_body(k, c2, src=src, out=out, full=full):
                cid = k * _NW + wid

                @pl.when(cid < full)
                def _():
                    pltpu.sync_copy(src.at[pl.ds(cid * nb, nb)], stage3)

                    def b_body(b, c3):
                        for sub in range(8):
                            rbase = b * 128 + sub * 16
                            f0v = stage3[b, 0, pl.ds(sub * 16, 16)]
                            f1v = stage3[b, 1, pl.ds(sub * 16, 16)]
                            _vscatter(chunk8, [rbase + iota, zero16], f0v)
                            _vscatter(chunk8, [rbase + iota, one16], f1v)
                        return c3
                    lax.fori_loop(0, nb, b_body, 0)
                    pltpu.sync_copy(chunk8, out.at[pl.ds(cid * _PCH, _PCH), :])
                return c2
            lax.fori_loop(0, -(-full // _NW), k_body, 0)
            continue
        if full:
            def k_body(k, c2, src=src, out=out, full=full):
                cid = k * _NW + wid

                @pl.when(cid < full)
                def _():
                    r0 = cid * _PCH
                    pltpu.sync_copy(src.at[pl.ds(r0 * 2, _PCH * 2)], stage)

                    def g_body(g, c3):
                        v = stage[pl.ds(g * 16, 16)]
                        _vscatter(chunk8, [rowi + g * 8, coli], v)
                        return c3
                    lax.fori_loop(0, _PCH // 8, g_body, 0)
                    pltpu.sync_copy(chunk8, out.at[pl.ds(r0, _PCH), :])
                return c2
            lax.fori_loop(0, -(-full // _NW), k_body, 0)
        if tail:
            @pl.when(wid == jnp.int32(lvl % _NW))
            def _(src=src, out=out, full=full, tail=tail):
                r0 = full * _PCH
                pltpu.sync_copy(src.at[pl.ds(r0 * 2, tail * 2)],
                                stage.at[pl.ds(0, tail * 2)])
                def g_body(g, c2):
                    v = stage[pl.ds(g * 16, 16)]
                    _vscatter(chunk8, [rowi + g * 8, coli], v)
                    return c2
                lax.fori_loop(0, tail // 8, g_body, 0)
                pltpu.sync_copy(chunk8.at[pl.ds(0, tail), :],
                                out.at[pl.ds(r0, tail), :])


_padder = functools.partial(
    pl.kernel,
    out_type=tuple(
        jax.ShapeDtypeStruct((_LVL[i][1], _ROW), jnp.float32)
        for i in range(_N_LEVELS)
    ),
    mesh=plsc.VectorSubcoreMesh(core_axis_name="c", subcore_axis_name="s",
                                num_cores=_NC, num_subcores=_NS),
    compiler_params=pltpu.CompilerParams(needs_layout_passes=False,
                                         use_tc_tiling_on_sc=False),
    scratch_types=[
        pltpu.VMEM((_PCH * 2,), jnp.float32),
        pltpu.VMEM((_PCH // 128, 2, 128), jnp.float32),
        pltpu.VMEM((_PCH, _ROW), jnp.float32),
    ],
)(_pad_body)


def _gather_rows(table, idx_ref, dst, sem):
    return pltpu.async_copy(table.at[idx_ref], dst, sem)


def _vgather(ref, idxs):
    return plsc.load_gather(ref, idxs)


def _vscatter(ref, idxs, x):
    plsc.store_scatter(ref, idxs, x)


def _worker_id():
    return lax.axis_index("s") * _NC + lax.axis_index("c")


def _body(xs_h, ys_h, zs_h, *rest):
    tables = rest[:_N_LEVELS]
    out_h = rest[_N_LEVELS]
    (xv, yv, zv, dxb, dyb, dzb, idxb, rowsb, outc, sem) = rest[_N_LEVELS + 1:]

    wid = _worker_id()
    base0 = wid * _PTS_PER_W
    iota = lax.iota(jnp.int32, 16)
    zero16 = jnp.zeros((16,), jnp.int32)
    one16 = jnp.ones((16,), jnp.int32)

    def chunk_body(ci, carry):
        base = base0 + ci * _C
        pltpu.sync_copy(xs_h.at[pl.ds(base, _C)], xv)
        pltpu.sync_copy(ys_h.at[pl.ds(base, _C)], yv)
        pltpu.sync_copy(zs_h.at[pl.ds(base, _C)], zv)

        copies = []
        for lvl in range(_N_LEVELS):
            scale, size, res, is_hash = _LVL[lvl]
            fscale = jnp.float32(scale)

            def body_a(g, c2, lvl=lvl, fscale=fscale, size=size, res=res,
                       is_hash=is_hash):
                off = g * 16
                xw = xv[pl.ds(off, 16)] * fscale + jnp.float32(0.5)
                yw = yv[pl.ds(off, 16)] * fscale + jnp.float32(0.5)
                zw = zv[pl.ds(off, 16)] * fscale + jnp.float32(0.5)
                xg = xw.astype(jnp.int32)
                yg = yw.astype(jnp.int32)
                zg = zw.astype(jnp.int32)
                dxb[lvl, pl.ds(off, 16)] = xw - xg.astype(jnp.float32)
                dyb[lvl, pl.ds(off, 16)] = yw - yg.astype(jnp.float32)
                dzb[lvl, pl.ds(off, 16)] = zw - zg.astype(jnp.float32)
                if is_hash:
                    mask = jnp.int32(size - 1)
                    hx = (xg, xg + jnp.int32(1))
                    hy0 = yg * _P2
                    hz0 = zg * _P3
                    hy = (hy0, hy0 + _P2)
                    hz = (hz0, hz0 + _P3)
                    for c, (ox, oy, oz) in enumerate(_CORNERS):
                        idxb[lvl, c, pl.ds(off, 16)] = (hx[ox] ^ hy[oy] ^ hz[oz]) & mask
                else:
                    s = jnp.int32(res)
                    s2 = jnp.int32(res * res)
                    tx = (xg, xg + jnp.int32(1))
                    ty0 = yg * s
                    tz0 = zg * s2
                    ty = (ty0, ty0 + s)
                    tz = (tz0, tz0 + s2)
                    sz = jnp.int32(size)
                    for c, (ox, oy, oz) in enumerate(_CORNERS):
                        idx = tx[ox] + ty[oy] + tz[oz]
                        idxb[lvl, c, pl.ds(off, 16)] = jnp.maximum(lax.rem(idx, sz), 0)
                return c2

            lax.fori_loop(0, _NG, body_a, 0)
            for c in range(8):
                copies.append(
                    _gather_rows(tables[lvl], idxb.at[lvl, c],
                                 rowsb.at[lvl, c], sem))
        for cp in copies:
            cp.wait()

        for lvl in range(_N_LEVELS):
            def body_b(g, c2, lvl=lvl):
                off = g * 16
                dx = dxb[lvl, pl.ds(off, 16)]
                dy = dyb[lvl, pl.ds(off, 16)]
                dz = dzb[lvl, pl.ds(off, 16)]
                one = jnp.float32(1.0)
                wx = (one - dx, dx)
                wy = (one - dy, dy)
                wz = (one - dz, dz)
                wxy = ((wx[0] * wy[0], wx[0] * wy[1]),
                       (wx[1] * wy[0], wx[1] * wy[1]))
                rows = iota + off
                acc0 = jnp.zeros((16,), jnp.float32)
                acc1 = jnp.zeros((16,), jnp.float32)
                for c, (ox, oy, oz) in enumerate(_CORNERS):
                    w = wxy[ox][oy] * wz[oz]
                    f0 = _vgather(rowsb.at[lvl, c], [rows, zero16])
                    f1 = _vgather(rowsb.at[lvl, c], [rows, one16])
                    acc0 = acc0 + f0 * w
                    acc1 = acc1 + f1 * w
                flat0 = rows * jnp.int32(_OUT_W) + jnp.int32(2 * lvl)
                _vscatter(outc, [flat0], acc0)
                _vscatter(outc, [flat0 + one16], acc1)
                return c2

            lax.fori_loop(0, _NG, body_b, 0)

        pltpu.sync_copy(outc, out_h.at[pl.ds(base * _OUT_W, _C * _OUT_W)])
        return carry

    lax.fori_loop(0, _CHUNKS, chunk_body, 0)


_encode = functools.partial(
    pl.kernel,
    out_type=jax.ShapeDtypeStruct((_N * _OUT_W,), jnp.float32),
    mesh=plsc.VectorSubcoreMesh(core_axis_name="c", subcore_axis_name="s",
                                num_cores=_NC, num_subcores=_NS),
    compiler_params=pltpu.CompilerParams(needs_layout_passes=False,
                                         use_tc_tiling_on_sc=False),
    scratch_types=[
        pltpu.VMEM((_C,), jnp.float32),
        pltpu.VMEM((_C,), jnp.float32),
        pltpu.VMEM((_C,), jnp.float32),
        pltpu.VMEM((_N_LEVELS, _C), jnp.float32),
        pltpu.VMEM((_N_LEVELS, _C), jnp.float32),
        pltpu.VMEM((_N_LEVELS, _C), jnp.float32),
        pltpu.VMEM((_N_LEVELS, 8, _C), jnp.int32),
        pltpu.VMEM((_N_LEVELS, 8, _C, _ROW), jnp.float32),
        pltpu.VMEM((_C * _OUT_W,), jnp.float32),
        pltpu.SemaphoreType.DMA,
    ],
)(_body)


def kernel(inputs, grids):
    xs = inputs[:, 0]
    ys = inputs[:, 1]
    zs = inputs[:, 2]
    flat = []
    for g in grids:
        s = g.shape[0]
        if s % 128 == 0:
            flat.append(g.reshape(s // 128, 128, 2).transpose(0, 2, 1))
        else:
            flat.append(g.reshape(-1))
    padded = _padder(*flat)
    flat = _encode(xs, ys, zs, *padded)
    return flat.reshape(_N, _OUT_W)


# speedup vs baseline: 11.6176x; 1.0177x over previous
import functools
import math

import numpy as np

import jax
import jax.numpy as jnp
from jax import lax
from jax.experimental import pallas as pl
from jax.experimental.pallas import tpu as pltpu
from jax.experimental.pallas import tpu_sc as plsc

_N_LEVELS = 16
_N_FEAT = 2
_ROW = 8
_LOG2_HASHMAP = 19
_BASE_RES = 16
_PER_LEVEL_SCALE = 1.38191288

_N = 262144
_NC = 2
_NS = 16
_NW = _NC * _NS
_C = 64
_NG = _C // 16
_PTS_PER_W = _N // _NW
_CHUNKS = _PTS_PER_W // _C
_OUT_W = 2 * _N_LEVELS

_P2 = np.int32(-1640531535)
_P3 = np.int32(805459861)

_CORNERS = [(ox, oy, oz) for ox in (0, 1) for oy in (0, 1) for oz in (0, 1)]


def _level_params():
    thr = 1 << _LOG2_HASHMAP
    params = []
    for i in range(_N_LEVELS):
        scale = math.pow(2.0, i * math.log2(_PER_LEVEL_SCALE)) * _BASE_RES - 1.0
        res = math.ceil(scale) + 1
        size = min(math.ceil(res ** 3 / 8) * 8, thr)
        params.append((scale, size, res, size >= thr))
    return params


_LVL = _level_params()


_PCH = 2048


def _pad_body(*args):
    srcs = args[:_N_LEVELS]
    outs = args[_N_LEVELS:2 * _N_LEVELS]
    stage, stage3, chunk8, = args[2 * _N_LEVELS:]
    wid = _worker_id()
    iota = lax.iota(jnp.int32, 16)
    rowi = lax.shift_right_logical(iota, 1)
    coli = lax.bitwise_and(iota, jnp.int32(1))
    zero16 = jnp.zeros((16,), jnp.int32)
    one16 = jnp.ones((16,), jnp.int32)
    nb = _PCH // 128

    for lvl in range(_N_LEVELS):
        size = _LVL[lvl][1]
        full = size // _PCH
        tail = size - full * _PCH
        src, out = srcs[lvl], outs[lvl]
        if size % 128 == 0:
            assert tail == 0 and full > 0
            def k_body(k, c2, src=src, out=out, full=full):
                cid = k * _NW + wid

                @pl.when(cid < full)
                def _():
                    pltpu.sync_copy(src.at[pl.ds(cid * nb, nb)], stage3)

                    def b_body(b, c3):
                        for sub in range(8):
                            rbase = b * 128 + sub * 16
                            f0v = stage3[b, 0, pl.ds(sub * 16, 16)]
                            f1v = stage3[b, 1, pl.ds(sub * 16, 16)]
                            _vscatter(chunk8, [rbase + iota, zero16], f0v)
                            _vscatter(chunk8, [rbase + iota, one16], f1v)
                        return c3
                    lax.fori_loop(0, nb, b_body, 0)
                    pltpu.sync_copy(chunk8, out.at[pl.ds(cid * _PCH, _PCH), :])
                return c2
            lax.fori_loop(0, -(-full // _NW), k_body, 0)
            continue
        if full:
            def k_body(k, c2, src=src, out=out, full=full):
                cid = k * _NW + wid

                @pl.when(cid < full)
                def _():
                    r0 = cid * _PCH
                    pltpu.sync_copy(src.at[pl.ds(r0 * 2, _PCH * 2)], stage)

                    def g_body(g, c3):
                        v = stage[pl.ds(g * 16, 16)]
                        _vscatter(chunk8, [rowi + g * 8, coli], v)
                        return c3
                    lax.fori_loop(0, _PCH // 8, g_body, 0)
                    pltpu.sync_copy(chunk8, out.at[pl.ds(r0, _PCH), :])
                return c2
            lax.fori_loop(0, -(-full // _NW), k_body, 0)
        if tail:
            @pl.when(wid == jnp.int32(lvl % _NW))
            def _(src=src, out=out, full=full, tail=tail):
                r0 = full * _PCH
                pltpu.sync_copy(src.at[pl.ds(r0 * 2, tail * 2)],
                                stage.at[pl.ds(0, tail * 2)])
                def g_body(g, c2):
                    v = stage[pl.ds(g * 16, 16)]
                    _vscatter(chunk8, [rowi + g * 8, coli], v)
                    return c2
                lax.fori_loop(0, tail // 8, g_body, 0)
                pltpu.sync_copy(chunk8.at[pl.ds(0, tail), :],
                                out.at[pl.ds(r0, tail), :])


_padder = functools.partial(
    pl.kernel,
    out_type=tuple(
        jax.ShapeDtypeStruct((_LVL[i][1], _ROW), jnp.float32)
        for i in range(_N_LEVELS)
    ),
    mesh=plsc.VectorSubcoreMesh(core_axis_name="c", subcore_axis_name="s",
                                num_cores=_NC, num_subcores=_NS),
    compiler_params=pltpu.CompilerParams(needs_layout_passes=False,
                                         use_tc_tiling_on_sc=False),
    scratch_types=[
        pltpu.VMEM((_PCH * 2,), jnp.float32),
        pltpu.VMEM((_PCH // 128, 2, 128), jnp.float32),
        pltpu.VMEM((_PCH, _ROW), jnp.float32),
    ],
)(_pad_body)


def _gather_rows(table, idx_ref, dst, sem):
    return pltpu.async_copy(table.at[idx_ref], dst, sem)


def _vgather(ref, idxs):
    return plsc.load_gather(ref, idxs)


def _vscatter(ref, idxs, x):
    plsc.store_scatter(ref, idxs, x)


def _worker_id():
    return lax.axis_index("s") * _NC + lax.axis_index("c")


def _body(xs_h, ys_h, zs_h, *rest):
    tables = rest[:_N_LEVELS]
    out_h = rest[_N_LEVELS]
    (xv, yv, zv, dxb, dyb, dzb, idxb, rowsb, outc, sem) = rest[_N_LEVELS + 1:]

    wid = _worker_id()
    base0 = wid * _PTS_PER_W
    iota = lax.iota(jnp.int32, 16)
    zero16 = jnp.zeros((16,), jnp.int32)
    one16 = jnp.ones((16,), jnp.int32)

    def chunk_body(ci, carry):
        base = base0 + ci * _C
        pltpu.sync_copy(xs_h.at[pl.ds(base, _C)], xv)
        pltpu.sync_copy(ys_h.at[pl.ds(base, _C)], yv)
        pltpu.sync_copy(zs_h.at[pl.ds(base, _C)], zv)

        copies = []
        for lvl in range(_N_LEVELS):
            scale, size, res, is_hash = _LVL[lvl]
            fscale = jnp.float32(scale)

            def body_a(g, c2, lvl=lvl, fscale=fscale, size=size, res=res,
                       is_hash=is_hash):
                off = g * 16
                xw = xv[pl.ds(off, 16)] * fscale + jnp.float32(0.5)
                yw = yv[pl.ds(off, 16)] * fscale + jnp.float32(0.5)
                zw = zv[pl.ds(off, 16)] * fscale + jnp.float32(0.5)
                xg = xw.astype(jnp.int32)
                yg = yw.astype(jnp.int32)
                zg = zw.astype(jnp.int32)
                dxb[lvl, pl.ds(off, 16)] = xw - xg.astype(jnp.float32)
                dyb[lvl, pl.ds(off, 16)] = yw - yg.astype(jnp.float32)
                dzb[lvl, pl.ds(off, 16)] = zw - zg.astype(jnp.float32)
                if is_hash:
                    mask = jnp.int32(size - 1)
                    hx = (xg, xg + jnp.int32(1))
                    hy0 = yg * _P2
                    hz0 = zg * _P3
                    hy = (hy0, hy0 + _P2)
                    hz = (hz0, hz0 + _P3)
                    for c, (ox, oy, oz) in enumerate(_CORNERS):
                        idxb[lvl, pl.ds(c * _C + off, 16)] = (hx[ox] ^ hy[oy] ^ hz[oz]) & mask
                else:
                    s = jnp.int32(res)
                    s2 = jnp.int32(res * res)
                    tx = (xg, xg + jnp.int32(1))
                    ty0 = yg * s
                    tz0 = zg * s2
                    ty = (ty0, ty0 + s)
                    tz = (tz0, tz0 + s2)
                    sz = jnp.int32(size)
                    for c, (ox, oy, oz) in enumerate(_CORNERS):
                        idx = tx[ox] + ty[oy] + tz[oz]
                        idxb[lvl, pl.ds(c * _C + off, 16)] = jnp.maximum(lax.rem(idx, sz), 0)
                return c2

            lax.fori_loop(0, _NG, body_a, 0)
            copies.append(
                _gather_rows(tables[lvl], idxb.at[lvl], rowsb.at[lvl], sem))
        for cp in copies:
            cp.wait()

        for lvl in range(_N_LEVELS):
            def body_b(g, c2, lvl=lvl):
                off = g * 16
                dx = dxb[lvl, pl.ds(off, 16)]
                dy = dyb[lvl, pl.ds(off, 16)]
                dz = dzb[lvl, pl.ds(off, 16)]
                one = jnp.float32(1.0)
                wx = (one - dx, dx)
                wy = (one - dy, dy)
                wz = (one - dz, dz)
                wxy = ((wx[0] * wy[0], wx[0] * wy[1]),
                       (wx[1] * wy[0], wx[1] * wy[1]))
                rows = iota + off
                acc0 = jnp.zeros((16,), jnp.float32)
                acc1 = jnp.zeros((16,), jnp.float32)
                for c, (ox, oy, oz) in enumerate(_CORNERS):
                    w = wxy[ox][oy] * wz[oz]
                    crows = rows + c * _C
                    f0 = _vgather(rowsb.at[lvl], [crows, zero16])
                    f1 = _vgather(rowsb.at[lvl], [crows, one16])
                    acc0 = acc0 + f0 * w
                    acc1 = acc1 + f1 * w
                flat0 = rows * jnp.int32(_OUT_W) + jnp.int32(2 * lvl)
                _vscatter(outc, [flat0], acc0)
                _vscatter(outc, [flat0 + one16], acc1)
                return c2

            lax.fori_loop(0, _NG, body_b, 0)

        pltpu.sync_copy(outc, out_h.at[pl.ds(base * _OUT_W, _C * _OUT_W)])
        return carry

    lax.fori_loop(0, _CHUNKS, chunk_body, 0)


_encode = functools.partial(
    pl.kernel,
    out_type=jax.ShapeDtypeStruct((_N * _OUT_W,), jnp.float32),
    mesh=plsc.VectorSubcoreMesh(core_axis_name="c", subcore_axis_name="s",
                                num_cores=_NC, num_subcores=_NS),
    compiler_params=pltpu.CompilerParams(needs_layout_passes=False,
                                         use_tc_tiling_on_sc=False),
    scratch_types=[
        pltpu.VMEM((_C,), jnp.float32),
        pltpu.VMEM((_C,), jnp.float32),
        pltpu.VMEM((_C,), jnp.float32),
        pltpu.VMEM((_N_LEVELS, _C), jnp.float32),
        pltpu.VMEM((_N_LEVELS, _C), jnp.float32),
        pltpu.VMEM((_N_LEVELS, _C), jnp.float32),
        pltpu.VMEM((_N_LEVELS, 8 * _C), jnp.int32),
        pltpu.VMEM((_N_LEVELS, 8 * _C, _ROW), jnp.float32),
        pltpu.VMEM((_C * _OUT_W,), jnp.float32),
        pltpu.SemaphoreType.DMA,
    ],
)(_body)


def kernel(inputs, grids):
    xs = inputs[:, 0]
    ys = inputs[:, 1]
    zs = inputs[:, 2]
    flat = []
    for g in grids:
        s = g.shape[0]
        if s % 128 == 0:
            flat.append(g.reshape(s // 128, 128, 2).transpose(0, 2, 1))
        else:
            flat.append(g.reshape(-1))
    padded = _padder(*flat)
    flat = _encode(xs, ys, zs, *padded)
    return flat.reshape(_N, _OUT_W)


# speedup vs baseline: 14.6979x; 1.2651x over previous
import functools
import math

import numpy as np

import jax
import jax.numpy as jnp
from jax import lax
from jax.experimental import pallas as pl
from jax.experimental.pallas import tpu as pltpu
from jax.experimental.pallas import tpu_sc as plsc

_N_LEVELS = 16
_N_FEAT = 2
_ROW = 8
_LOG2_HASHMAP = 19
_BASE_RES = 16
_PER_LEVEL_SCALE = 1.38191288

_N = 262144
_NC = 2
_NS = 16
_NW = _NC * _NS
_C = 64
_NG = _C // 16
_PTS_PER_W = _N // _NW
_CHUNKS = _PTS_PER_W // _C
_OUT_W = 2 * _N_LEVELS

_P2 = np.int32(-1640531535)
_P3 = np.int32(805459861)

_CORNERS = [(ox, oy, oz) for ox in (0, 1) for oy in (0, 1) for oz in (0, 1)]


def _level_params():
    thr = 1 << _LOG2_HASHMAP
    params = []
    for i in range(_N_LEVELS):
        scale = math.pow(2.0, i * math.log2(_PER_LEVEL_SCALE)) * _BASE_RES - 1.0
        res = math.ceil(scale) + 1
        size = min(math.ceil(res ** 3 / 8) * 8, thr)
        params.append((scale, size, res, size >= thr))
    return params


_LVL = _level_params()


_PCH = 2048


def _pad_body(*args):
    srcs = args[:_N_LEVELS]
    outs = args[_N_LEVELS:2 * _N_LEVELS]
    stage, stage3, chunk8, = args[2 * _N_LEVELS:]
    wid = _worker_id()
    iota = lax.iota(jnp.int32, 16)
    rowi = lax.shift_right_logical(iota, 1)
    coli = lax.bitwise_and(iota, jnp.int32(1))
    zero16 = jnp.zeros((16,), jnp.int32)
    one16 = jnp.ones((16,), jnp.int32)
    nb = _PCH // 128

    for lvl in range(_N_LEVELS):
        size = _LVL[lvl][1]
        full = size // _PCH
        tail = size - full * _PCH
        src, out = srcs[lvl], outs[lvl]
        if size % 128 == 0:
            assert tail == 0 and full > 0
            def k_body(k, c2, src=src, out=out, full=full):
                cid = k * _NW + wid

                @pl.when(cid < full)
                def _():
                    pltpu.sync_copy(src.at[pl.ds(cid * nb, nb)], stage3)

                    def b_body(b, c3):
                        for sub in range(8):
                            rbase = b * 128 + sub * 16
                            f0v = stage3[b, 0, pl.ds(sub * 16, 16)]
                            f1v = stage3[b, 1, pl.ds(sub * 16, 16)]
                            _vscatter(chunk8, [rbase + iota, zero16], f0v)
                            _vscatter(chunk8, [rbase + iota, one16], f1v)
                        return c3
                    lax.fori_loop(0, nb, b_body, 0)
                    pltpu.sync_copy(chunk8, out.at[pl.ds(cid * _PCH, _PCH), :])
                return c2
            lax.fori_loop(0, -(-full // _NW), k_body, 0)
            continue
        if full:
            def k_body(k, c2, src=src, out=out, full=full):
                cid = k * _NW + wid

                @pl.when(cid < full)
                def _():
                    r0 = cid * _PCH
                    pltpu.sync_copy(src.at[pl.ds(r0 * 2, _PCH * 2)], stage)

                    def g_body(g, c3):
                        v = stage[pl.ds(g * 16, 16)]
                        _vscatter(chunk8, [rowi + g * 8, coli], v)
                        return c3
                    lax.fori_loop(0, _PCH // 8, g_body, 0)
                    pltpu.sync_copy(chunk8, out.at[pl.ds(r0, _PCH), :])
                return c2
            lax.fori_loop(0, -(-full // _NW), k_body, 0)
        if tail:
            @pl.when(wid == jnp.int32(lvl % _NW))
            def _(src=src, out=out, full=full, tail=tail):
                r0 = full * _PCH
                pltpu.sync_copy(src.at[pl.ds(r0 * 2, tail * 2)],
                                stage.at[pl.ds(0, tail * 2)])
                def g_body(g, c2):
                    v = stage[pl.ds(g * 16, 16)]
                    _vscatter(chunk8, [rowi + g * 8, coli], v)
                    return c2
                lax.fori_loop(0, tail // 8, g_body, 0)
                pltpu.sync_copy(chunk8.at[pl.ds(0, tail), :],
                                out.at[pl.ds(r0, tail), :])


_padder = functools.partial(
    pl.kernel,
    out_type=tuple(
        jax.ShapeDtypeStruct((_LVL[i][1], _ROW), jnp.float32)
        for i in range(_N_LEVELS)
    ),
    mesh=plsc.VectorSubcoreMesh(core_axis_name="c", subcore_axis_name="s",
                                num_cores=_NC, num_subcores=_NS),
    compiler_params=pltpu.CompilerParams(needs_layout_passes=False,
                                         use_tc_tiling_on_sc=False),
    scratch_types=[
        pltpu.VMEM((_PCH * 2,), jnp.float32),
        pltpu.VMEM((_PCH // 128, 2, 128), jnp.float32),
        pltpu.VMEM((_PCH, _ROW), jnp.float32),
    ],
)(_pad_body)


def _gather_rows(table, idx_ref, dst, sem):
    return pltpu.async_copy(table.at[idx_ref], dst, sem)


def _wait_rows(table, idx_ref, dst, sem):
    pltpu.make_async_copy(table.at[idx_ref], dst, sem).wait()


def _vgather(ref, idxs):
    return plsc.load_gather(ref, idxs)


def _vscatter(ref, idxs, x):
    plsc.store_scatter(ref, idxs, x)


def _worker_id():
    return lax.axis_index("s") * _NC + lax.axis_index("c")


_H = _N_LEVELS // 2
_H0 = tuple(range(_H))
_H1 = tuple(range(_H, _N_LEVELS))


def _body(xs_h, ys_h, zs_h, *rest):
    tables = rest[:_N_LEVELS]
    out_h = rest[_N_LEVELS]
    (xv, yv, zv, dxb, dyb, dzb, idxb, rowsb, outc, sem0, sem1) = rest[_N_LEVELS + 1:]

    wid = _worker_id()
    base0 = wid * _PTS_PER_W
    iota = lax.iota(jnp.int32, 16)
    zero16 = jnp.zeros((16,), jnp.int32)
    one16 = jnp.ones((16,), jnp.int32)

    def load_coords(ci):
        base = base0 + ci * _C
        pltpu.sync_copy(xs_h.at[pl.ds(base, _C)], xv)
        pltpu.sync_copy(ys_h.at[pl.ds(base, _C)], yv)
        pltpu.sync_copy(zs_h.at[pl.ds(base, _C)], zv)

    def phase_a(levels, sem):
        for lvl in levels:
            scale, size, res, is_hash = _LVL[lvl]
            fscale = jnp.float32(scale)

            def body_a(g, c2, lvl=lvl, fscale=fscale, size=size, res=res,
                       is_hash=is_hash):
                off = g * 16
                xw = xv[pl.ds(off, 16)] * fscale + jnp.float32(0.5)
                yw = yv[pl.ds(off, 16)] * fscale + jnp.float32(0.5)
                zw = zv[pl.ds(off, 16)] * fscale + jnp.float32(0.5)
                xg = xw.astype(jnp.int32)
                yg = yw.astype(jnp.int32)
                zg = zw.astype(jnp.int32)
                dxb[lvl, pl.ds(off, 16)] = xw - xg.astype(jnp.float32)
                dyb[lvl, pl.ds(off, 16)] = yw - yg.astype(jnp.float32)
                dzb[lvl, pl.ds(off, 16)] = zw - zg.astype(jnp.float32)
                if is_hash:
                    mask = jnp.int32(size - 1)
                    hx = (xg, xg + jnp.int32(1))
                    hy0 = yg * _P2
                    hz0 = zg * _P3
                    hy = (hy0, hy0 + _P2)
                    hz = (hz0, hz0 + _P3)
                    for c, (ox, oy, oz) in enumerate(_CORNERS):
                        idxb[lvl, pl.ds(c * _C + off, 16)] = (hx[ox] ^ hy[oy] ^ hz[oz]) & mask
                else:
                    s = jnp.int32(res)
                    s2 = jnp.int32(res * res)
                    tx = (xg, xg + jnp.int32(1))
                    ty0 = yg * s
                    tz0 = zg * s2
                    ty = (ty0, ty0 + s)
                    tz = (tz0, tz0 + s2)
                    sz = jnp.int32(size)
                    for c, (ox, oy, oz) in enumerate(_CORNERS):
                        idx = tx[ox] + ty[oy] + tz[oz]
                        idxb[lvl, pl.ds(c * _C + off, 16)] = jnp.maximum(lax.rem(idx, sz), 0)
                return c2

            lax.fori_loop(0, _NG, body_a, 0)
            _gather_rows(tables[lvl], idxb.at[lvl], rowsb.at[lvl], sem)

    def wait_half(levels, sem):
        for lvl in levels:
            _wait_rows(tables[lvl], idxb.at[lvl], rowsb.at[lvl], sem)

    def phase_b(levels):
        for lvl in levels:
            def body_b(g, c2, lvl=lvl):
                off = g * 16
                dx = dxb[lvl, pl.ds(off, 16)]
                dy = dyb[lvl, pl.ds(off, 16)]
                dz = dzb[lvl, pl.ds(off, 16)]
                one = jnp.float32(1.0)
                wx = (one - dx, dx)
                wy = (one - dy, dy)
                wz = (one - dz, dz)
                wxy = ((wx[0] * wy[0], wx[0] * wy[1]),
                       (wx[1] * wy[0], wx[1] * wy[1]))
                rows = iota + off
                acc0 = jnp.zeros((16,), jnp.float32)
                acc1 = jnp.zeros((16,), jnp.float32)
                for c, (ox, oy, oz) in enumerate(_CORNERS):
                    w = wxy[ox][oy] * wz[oz]
                    crows = rows + c * _C
                    f0 = _vgather(rowsb.at[lvl], [crows, zero16])
                    f1 = _vgather(rowsb.at[lvl], [crows, one16])
                    acc0 = acc0 + f0 * w
                    acc1 = acc1 + f1 * w
                flat0 = rows * jnp.int32(_OUT_W) + jnp.int32(2 * lvl)
                _vscatter(outc, [flat0], acc0)
                _vscatter(outc, [flat0 + one16], acc1)
                return c2

            lax.fori_loop(0, _NG, body_b, 0)

    load_coords(0)
    phase_a(_H0, sem0)

    def chunk_body(ci, carry):
        phase_a(_H1, sem1)
        wait_half(_H0, sem0)
        phase_b(_H0)

        @pl.when(ci + 1 < _CHUNKS)
        def _():
            load_coords(ci + 1)
            phase_a(_H0, sem0)

        wait_half(_H1, sem1)
        phase_b(_H1)
        base = base0 + ci * _C
        pltpu.sync_copy(outc, out_h.at[pl.ds(base * _OUT_W, _C * _OUT_W)])
        return carry

    lax.fori_loop(0, _CHUNKS, chunk_body, 0)


_encode = functools.partial(
    pl.kernel,
    out_type=jax.ShapeDtypeStruct((_N * _OUT_W,), jnp.float32),
    mesh=plsc.VectorSubcoreMesh(core_axis_name="c", subcore_axis_name="s",
                                num_cores=_NC, num_subcores=_NS),
    compiler_params=pltpu.CompilerParams(needs_layout_passes=False,
                                         use_tc_tiling_on_sc=False),
    scratch_types=[
        pltpu.VMEM((_C,), jnp.float32),
        pltpu.VMEM((_C,), jnp.float32),
        pltpu.VMEM((_C,), jnp.float32),
        pltpu.VMEM((_N_LEVELS, _C), jnp.float32),
        pltpu.VMEM((_N_LEVELS, _C), jnp.float32),
        pltpu.VMEM((_N_LEVELS, _C), jnp.float32),
        pltpu.VMEM((_N_LEVELS, 8 * _C), jnp.int32),
        pltpu.VMEM((_N_LEVELS, 8 * _C, _ROW), jnp.float32),
        pltpu.VMEM((_C * _OUT_W,), jnp.float32),
        pltpu.SemaphoreType.DMA,
        pltpu.SemaphoreType.DMA,
    ],
)(_body)


def kernel(inputs, grids):
    xs = inputs[:, 0]
    ys = inputs[:, 1]
    zs = inputs[:, 2]
    flat = []
    for g in grids:
        s = g.shape[0]
        if s % 128 == 0:
            flat.append(g.reshape(s // 128, 128, 2).transpose(0, 2, 1))
        else:
            flat.append(g.reshape(-1))
    padded = _padder(*flat)
    flat = _encode(xs, ys, zs, *padded)
    return flat.reshape(_N, _OUT_W)


# speedup vs baseline: 15.9092x; 1.0824x over previous
import functools
import math

import numpy as np

import jax
import jax.numpy as jnp
from jax import lax
from jax.experimental import pallas as pl
from jax.experimental.pallas import tpu as pltpu
from jax.experimental.pallas import tpu_sc as plsc

_N_LEVELS = 16
_N_FEAT = 2
_ROW = 8
_LOG2_HASHMAP = 19
_BASE_RES = 16
_PER_LEVEL_SCALE = 1.38191288

_N = 262144
_NC = 2
_NS = 16
_NW = _NC * _NS
_C = 64
_NG = _C // 16
_PTS_PER_W = _N // _NW
_CHUNKS = _PTS_PER_W // _C
_OUT_W = 2 * _N_LEVELS

_P2 = np.int32(-1640531535)
_P3 = np.int32(805459861)

_CORNERS = [(ox, oy, oz) for ox in (0, 1) for oy in (0, 1) for oz in (0, 1)]


def _level_params():
    thr = 1 << _LOG2_HASHMAP
    params = []
    for i in range(_N_LEVELS):
        scale = math.pow(2.0, i * math.log2(_PER_LEVEL_SCALE)) * _BASE_RES - 1.0
        res = math.ceil(scale) + 1
        size = min(math.ceil(res ** 3 / 8) * 8, thr)
        params.append((scale, size, res, size >= thr))
    return params


_LVL = _level_params()


_PCH = 2048


def _pad_body(*args):
    srcs = args[:_N_LEVELS]
    outs = args[_N_LEVELS:2 * _N_LEVELS]
    stage, stage3, chunk8, = args[2 * _N_LEVELS:]
    wid = _worker_id()
    iota = lax.iota(jnp.int32, 16)
    rowi = lax.shift_right_logical(iota, 1)
    coli = lax.bitwise_and(iota, jnp.int32(1))
    zero16 = jnp.zeros((16,), jnp.int32)
    one16 = jnp.ones((16,), jnp.int32)
    nb = _PCH // 128

    for lvl in range(_N_LEVELS):
        size = _LVL[lvl][1]
        full = size // _PCH
        tail = size - full * _PCH
        src, out = srcs[lvl], outs[lvl]
        if size % 128 == 0:
            assert tail == 0 and full > 0
            def k_body(k, c2, src=src, out=out, full=full):
                cid = k * _NW + wid

                @pl.when(cid < full)
                def _():
                    pltpu.sync_copy(src.at[pl.ds(cid * nb, nb)], stage3)

                    def b_body(b, c3):
                        for sub in range(8):
                            rbase = b * 128 + sub * 16
                            f0v = stage3[b, 0, pl.ds(sub * 16, 16)]
                            f1v = stage3[b, 1, pl.ds(sub * 16, 16)]
                            _vscatter(chunk8, [rbase + iota, zero16], f0v)
                            _vscatter(chunk8, [rbase + iota, one16], f1v)
                        return c3
                    lax.fori_loop(0, nb, b_body, 0)
                    pltpu.sync_copy(chunk8, out.at[pl.ds(cid * _PCH, _PCH), :])
                return c2
            lax.fori_loop(0, -(-full // _NW), k_body, 0)
            continue
        if full:
            def k_body(k, c2, src=src, out=out, full=full):
                cid = k * _NW + wid

                @pl.when(cid < full)
                def _():
                    r0 = cid * _PCH
                    pltpu.sync_copy(src.at[pl.ds(r0 * 2, _PCH * 2)], stage)

                    def g_body(g, c3):
                        v = stage[pl.ds(g * 16, 16)]
                        _vscatter(chunk8, [rowi + g * 8, coli], v)
                        return c3
                    lax.fori_loop(0, _PCH // 8, g_body, 0)
                    pltpu.sync_copy(chunk8, out.at[pl.ds(r0, _PCH), :])
                return c2
            lax.fori_loop(0, -(-full // _NW), k_body, 0)
        if tail:
            @pl.when(wid == jnp.int32(lvl % _NW))
            def _(src=src, out=out, full=full, tail=tail):
                r0 = full * _PCH
                pltpu.sync_copy(src.at[pl.ds(r0 * 2, tail * 2)],
                                stage.at[pl.ds(0, tail * 2)])
                def g_body(g, c2):
                    v = stage[pl.ds(g * 16, 16)]
                    _vscatter(chunk8, [rowi + g * 8, coli], v)
                    return c2
                lax.fori_loop(0, tail // 8, g_body, 0)
                pltpu.sync_copy(chunk8.at[pl.ds(0, tail), :],
                                out.at[pl.ds(r0, tail), :])


_padder = functools.partial(
    pl.kernel,
    out_type=tuple(
        jax.ShapeDtypeStruct((_LVL[i][1], _ROW), jnp.float32)
        for i in range(_N_LEVELS)
    ),
    mesh=plsc.VectorSubcoreMesh(core_axis_name="c", subcore_axis_name="s",
                                num_cores=_NC, num_subcores=_NS),
    compiler_params=pltpu.CompilerParams(needs_layout_passes=False,
                                         use_tc_tiling_on_sc=False),
    scratch_types=[
        pltpu.VMEM((_PCH * 2,), jnp.float32),
        pltpu.VMEM((_PCH // 128, 2, 128), jnp.float32),
        pltpu.VMEM((_PCH, _ROW), jnp.float32),
    ],
)(_pad_body)


def _gather_rows(table, idx_ref, dst, sem):
    return pltpu.async_copy(table.at[idx_ref], dst, sem)


def _wait_rows(table, idx_ref, dst, sem):
    pltpu.make_async_copy(table.at[idx_ref], dst, sem).wait()


def _vgather(ref, idxs):
    return plsc.load_gather(ref, idxs)


def _vscatter(ref, idxs, x):
    plsc.store_scatter(ref, idxs, x)


def _worker_id():
    return lax.axis_index("s") * _NC + lax.axis_index("c")


_H = _N_LEVELS // 2
_H0 = tuple(range(_H))
_H1 = tuple(range(_H, _N_LEVELS))


def _body(xs_h, ys_h, zs_h, *rest):
    tables = rest[:_N_LEVELS]
    out_h = rest[_N_LEVELS]
    (xyzv, dxb, dyb, dzb, idxb, rowsb, outc, sem0, sem1, semc, semo) = rest[_N_LEVELS + 1:]

    wid = _worker_id()
    base0 = wid * _PTS_PER_W
    iota = lax.iota(jnp.int32, 16)
    zero16 = jnp.zeros((16,), jnp.int32)
    one16 = jnp.ones((16,), jnp.int32)

    def coord_copies(ci, bi):
        base = base0 + ci * _C
        return (
            (xs_h.at[pl.ds(base, _C)], xyzv.at[bi, 0]),
            (ys_h.at[pl.ds(base, _C)], xyzv.at[bi, 1]),
            (zs_h.at[pl.ds(base, _C)], xyzv.at[bi, 2]),
        )

    def start_coords(ci, bi):
        for src, dst in coord_copies(ci, bi):
            pltpu.async_copy(src, dst, semc)

    def wait_coords(ci, bi):
        for src, dst in coord_copies(ci, bi):
            pltpu.make_async_copy(src, dst, semc).wait()

    def phase_a(levels, sem, bi):
        for lvl in levels:
            scale, size, res, is_hash = _LVL[lvl]
            fscale = jnp.float32(scale)

            def body_a(g, c2, lvl=lvl, fscale=fscale, size=size, res=res,
                       is_hash=is_hash):
                off = g * 16
                xw = xyzv[bi, 0, pl.ds(off, 16)] * fscale + jnp.float32(0.5)
                yw = xyzv[bi, 1, pl.ds(off, 16)] * fscale + jnp.float32(0.5)
                zw = xyzv[bi, 2, pl.ds(off, 16)] * fscale + jnp.float32(0.5)
                xg = xw.astype(jnp.int32)
                yg = yw.astype(jnp.int32)
                zg = zw.astype(jnp.int32)
                dxb[lvl, pl.ds(off, 16)] = xw - xg.astype(jnp.float32)
                dyb[lvl, pl.ds(off, 16)] = yw - yg.astype(jnp.float32)
                dzb[lvl, pl.ds(off, 16)] = zw - zg.astype(jnp.float32)
                if is_hash:
                    mask = jnp.int32(size - 1)
                    hx = (xg, xg + jnp.int32(1))
                    hy0 = yg * _P2
                    hz0 = zg * _P3
                    hy = (hy0, hy0 + _P2)
                    hz = (hz0, hz0 + _P3)
                    for c, (ox, oy, oz) in enumerate(_CORNERS):
                        idxb[lvl, pl.ds(c * _C + off, 16)] = (hx[ox] ^ hy[oy] ^ hz[oz]) & mask
                else:
                    s = jnp.int32(res)
                    s2 = jnp.int32(res * res)
                    tx = (xg, xg + jnp.int32(1))
                    ty0 = yg * s
                    tz0 = zg * s2
                    ty = (ty0, ty0 + s)
                    tz = (tz0, tz0 + s2)
                    sz = jnp.int32(size)
                    for c, (ox, oy, oz) in enumerate(_CORNERS):
                        idx = tx[ox] + ty[oy] + tz[oz]
                        idxb[lvl, pl.ds(c * _C + off, 16)] = jnp.maximum(lax.rem(idx, sz), 0)
                return c2

            lax.fori_loop(0, _NG, body_a, 0)
            _gather_rows(tables[lvl], idxb.at[lvl], rowsb.at[lvl], sem)

    def wait_half(levels, sem):
        for lvl in levels:
            _wait_rows(tables[lvl], idxb.at[lvl], rowsb.at[lvl], sem)

    def phase_b(levels, ob):
        for lvl in levels:
            def body_b(g, c2, lvl=lvl):
                off = g * 16
                dx = dxb[lvl, pl.ds(off, 16)]
                dy = dyb[lvl, pl.ds(off, 16)]
                dz = dzb[lvl, pl.ds(off, 16)]
                one = jnp.float32(1.0)
                wx = (one - dx, dx)
                wy = (one - dy, dy)
                wz = (one - dz, dz)
                wxy = ((wx[0] * wy[0], wx[0] * wy[1]),
                       (wx[1] * wy[0], wx[1] * wy[1]))
                rows = iota + off
                acc0 = jnp.zeros((16,), jnp.float32)
                acc1 = jnp.zeros((16,), jnp.float32)
                for c, (ox, oy, oz) in enumerate(_CORNERS):
                    w = wxy[ox][oy] * wz[oz]
                    crows = rows + c * _C
                    f0 = _vgather(rowsb.at[lvl], [crows, zero16])
                    f1 = _vgather(rowsb.at[lvl], [crows, one16])
                    acc0 = acc0 + f0 * w
                    acc1 = acc1 + f1 * w
                flat0 = rows * jnp.int32(_OUT_W) + jnp.int32(2 * lvl)
                _vscatter(outc.at[ob], [flat0], acc0)
                _vscatter(outc.at[ob], [flat0 + one16], acc1)
                return c2

            lax.fori_loop(0, _NG, body_b, 0)

    def out_copy(ci, ob):
        base = base0 + ci * _C
        return (outc.at[ob], out_h.at[pl.ds(base * _OUT_W, _C * _OUT_W)])

    start_coords(0, 0)
    wait_coords(0, 0)
    phase_a(_H0, sem0, 0)

    def chunk_body(ci, carry):
        bi = ci & 1
        bn = 1 - bi

        @pl.when(ci + 1 < _CHUNKS)
        def _():
            start_coords(ci + 1, bn)

        phase_a(_H1, sem1, bi)

        @pl.when(ci >= 2)
        def _():
            src, dst = out_copy(ci - 2, bi)
            pltpu.make_async_copy(src, dst, semo).wait()

        wait_half(_H0, sem0)
        phase_b(_H0, bi)

        @pl.when(ci + 1 < _CHUNKS)
        def _():
            wait_coords(ci + 1, bn)
            phase_a(_H0, sem0, bn)

        wait_half(_H1, sem1)
        phase_b(_H1, bi)
        src, dst = out_copy(ci, bi)
        pltpu.async_copy(src, dst, semo)
        return carry

    lax.fori_loop(0, _CHUNKS, chunk_body, 0)
    for ci in (_CHUNKS - 2, _CHUNKS - 1):
        src, dst = out_copy(jnp.int32(ci), ci & 1)
        pltpu.make_async_copy(src, dst, semo).wait()


_encode = functools.partial(
    pl.kernel,
    out_type=jax.ShapeDtypeStruct((_N * _OUT_W,), jnp.float32),
    mesh=plsc.VectorSubcoreMesh(core_axis_name="c", subcore_axis_name="s",
                                num_cores=_NC, num_subcores=_NS),
    compiler_params=pltpu.CompilerParams(needs_layout_passes=False,
                                         use_tc_tiling_on_sc=False),
    scratch_types=[
        pltpu.VMEM((2, 3, _C), jnp.float32),
        pltpu.VMEM((_N_LEVELS, _C), jnp.float32),
        pltpu.VMEM((_N_LEVELS, _C), jnp.float32),
        pltpu.VMEM((_N_LEVELS, _C), jnp.float32),
        pltpu.VMEM((_N_LEVELS, 8 * _C), jnp.int32),
        pltpu.VMEM((_N_LEVELS, 8 * _C, _ROW), jnp.float32),
        pltpu.VMEM((2, _C * _OUT_W), jnp.float32),
        pltpu.SemaphoreType.DMA,
        pltpu.SemaphoreType.DMA,
        pltpu.SemaphoreType.DMA,
        pltpu.SemaphoreType.DMA,
    ],
)(_body)


def kernel(inputs, grids):
    xs = inputs[:, 0]
    ys = inputs[:, 1]
    zs = inputs[:, 2]
    flat = []
    for g in grids:
        s = g.shape[0]
        if s % 128 == 0:
            flat.append(g.reshape(s // 128, 128, 2).transpose(0, 2, 1))
        else:
            flat.append(g.reshape(-1))
    padded = _padder(*flat)
    flat = _encode(xs, ys, zs, *padded)
    return flat.reshape(_N, _OUT_W)
